# Initial kernel scaffold; baseline (speedup 1.0000x reference)
import jax
import jax.numpy as jnp
from jax.experimental import pallas as pl


def kernel(feats, xyz, index_0, index_1, index_0_offsets, n_max, qkv_w, qkv_b, proj_w, proj_b, rel_q_table, rel_k_table):
    raise NotImplementedError("write your pallas kernel here")


# speedup vs baseline: 3.7794x; 3.7794x over previous
import functools

import jax
import jax.numpy as jnp
from jax.experimental import pallas as pl
from jax.experimental.pallas import tpu as pltpu

_DIM = 128
_HEADS = 8
_HD = _DIM // _HEADS
_WINDOW_SIZE = 0.6
_QUANT_SIZE = 0.075
_GRID_LEN = int((2 * _WINDOW_SIZE + 0.0001) // _QUANT_SIZE)
_SCALE = _HD ** (-0.5)
_W = 128


def _smem():
    for name in ("SMEM",):
        v = getattr(pltpu, name, None)
        if v is not None:
            return v
    return pltpu.TPUMemorySpace.SMEM


def _pick_block(n, candidates):
    for c in candidates:
        if n % c == 0:
            return c
    return 1


def _linear_kern(x_ref, w_ref, b_ref, o_ref):
    o_ref[...] = (
        jnp.dot(x_ref[...], w_ref[...], preferred_element_type=jnp.float32)
        + b_ref[...]
    )


def _linear(x, w, b):
    n, k = x.shape
    m = w.shape[1]
    r = _pick_block(n, (2000, 1000, 500, 250, 100, 50, 10, 8, 2))
    return pl.pallas_call(
        _linear_kern,
        grid=(n // r,),
        in_specs=[
            pl.BlockSpec((r, k), lambda i: (i, 0)),
            pl.BlockSpec((k, m), lambda i: (0, 0)),
            pl.BlockSpec((1, m), lambda i: (0, 0)),
        ],
        out_specs=pl.BlockSpec((r, m), lambda i: (i, 0)),
        out_shape=jax.ShapeDtypeStruct((n, m), jnp.float32),
    )(x, w, b.reshape(1, m))


def _attn_kern(qb, nbr_ref, cnt_ref, qx_ref, kvx_ref, qtab_ref, ktab_ref,
               o_ref, kvbuf):
    f32 = jnp.float32
    g = (jax.lax.broadcasted_iota(jnp.int32, (_DIM, _HEADS), 0) // _HD
         == jax.lax.broadcasted_iota(jnp.int32, (_DIM, _HEADS), 1)).astype(f32)
    gt = (jax.lax.broadcasted_iota(jnp.int32, (_HEADS, _DIM), 1) // _HD
          == jax.lax.broadcasted_iota(jnp.int32, (_HEADS, _DIM), 0)).astype(f32)
    qtab = qtab_ref[...]
    ktab = ktab_ref[...]
    nbins = 2 * _GRID_LEN
    col = jax.lax.broadcasted_iota(jnp.int32, (_W, 3 * nbins), 1)
    col_c = col // nbins
    col_r = col % nbins
    tvec8 = jax.lax.broadcasted_iota(jnp.int32, (_W, _HEADS), 0)
    tvec128 = jax.lax.broadcasted_iota(jnp.int32, (_W, _DIM), 0)

    for qq in range(qb):
        cnt = jnp.minimum(cnt_ref[qq, 0], _W)

        def body(t, _, qq=qq):
            j = nbr_ref[qq, t]
            kvbuf[pl.ds(t, 1), :] = kvx_ref[pl.ds(j, 1), :]
            return 0

        jax.lax.fori_loop(0, cnt, body, 0)

        qrow = qx_ref[qq:qq + 1, 0:_DIM]
        xq = qx_ref[qq:qq + 1, _DIM:2 * _DIM]
        kb = kvbuf[:, 0:_DIM]
        vb = kvbuf[:, _DIM:2 * _DIM]
        xk = kvbuf[:, 2 * _DIM:3 * _DIM]

        rel = xq - xk
        rel = jnp.round(rel * 100000.0) / 100000.0
        ridx = jnp.floor(
            (rel + (2 * _WINDOW_SIZE - 0.0001)) / _QUANT_SIZE
        ).astype(jnp.int32)
        r0 = jax.lax.slice(ridx, (0, 0), (_W, 1))
        r1 = jax.lax.slice(ridx, (0, 1), (_W, 2))
        r2 = jax.lax.slice(ridx, (0, 2), (_W, 3))
        rsel = jnp.where(col_c == 0, r0, jnp.where(col_c == 1, r1, r2))
        onehot = (rsel == col_r).astype(f32)
        tqs = jnp.dot(onehot, qtab, preferred_element_type=f32)
        tks = jnp.dot(onehot, ktab, preferred_element_type=f32)

        combo = kb * qrow + tqs * qrow + tks * kb
        logits = jnp.dot(combo, g, preferred_element_type=f32)
        valid = tvec8 < cnt
        logits = jnp.where(valid, logits, jnp.float32(-1e30))
        mx = jnp.max(logits, axis=0, keepdims=True)
        e = jnp.where(valid, jnp.exp(logits - mx), 0.0)
        s = jnp.sum(e, axis=0, keepdims=True)
        s = jnp.where(s > 0.0, s, 1.0)
        soft = e / s
        soft128 = jnp.dot(soft, gt, preferred_element_type=f32)
        contrib = jnp.where(tvec128 < cnt, soft128 * vb, 0.0)
        o_ref[qq:qq + 1, :] = jnp.sum(contrib, axis=0, keepdims=True)


def kernel(feats, xyz, index_0, index_1, index_0_offsets, n_max,
           qkv_w, qkv_b, proj_w, proj_b, rel_q_table, rel_k_table):
    n, c = feats.shape
    m = index_1.shape[0]
    nbins = 2 * _GRID_LEN

    qkv = _linear(feats, qkv_w.T, qkv_b)
    qs = qkv[:, 0:_DIM] * _SCALE
    k = qkv[:, _DIM:2 * _DIM]
    v = qkv[:, 2 * _DIM:3 * _DIM]

    xyzp = jnp.concatenate(
        [xyz.astype(jnp.float32), jnp.zeros((n, _DIM - 3), jnp.float32)], axis=1)
    qx = jnp.concatenate([qs, xyzp], axis=1)
    kvx = jnp.concatenate([k, v, xyzp], axis=1)

    offs = index_0_offsets.astype(jnp.int32)
    counts = (offs[1:] - offs[:-1]).reshape(n, 1)
    ar = jnp.arange(_W, dtype=jnp.int32)
    pos = jnp.clip(offs[:-1, None] + ar[None, :], 0, m - 1)
    nbr = jnp.where(ar[None, :] < counts, index_1[pos], 0).astype(jnp.int32)

    qtab = rel_q_table.transpose(3, 0, 1, 2).reshape(3 * nbins, _DIM)
    ktab = rel_k_table.transpose(3, 0, 1, 2).reshape(3 * nbins, _DIM)
    qtab = qtab.astype(jnp.float32)
    ktab = ktab.astype(jnp.float32)

    qb = _pick_block(n, (8, 4, 2))
    smem = _smem()
    x = pl.pallas_call(
        functools.partial(_attn_kern, qb),
        grid=(n // qb,),
        in_specs=[
            pl.BlockSpec((qb, _W), lambda i: (i, 0), memory_space=smem),
            pl.BlockSpec((qb, 1), lambda i: (i, 0), memory_space=smem),
            pl.BlockSpec((qb, 2 * _DIM), lambda i: (i, 0)),
            pl.BlockSpec((n, 3 * _DIM), lambda i: (0, 0)),
            pl.BlockSpec((3 * nbins, _DIM), lambda i: (0, 0)),
            pl.BlockSpec((3 * nbins, _DIM), lambda i: (0, 0)),
        ],
        out_specs=pl.BlockSpec((qb, _DIM), lambda i: (i, 0)),
        out_shape=jax.ShapeDtypeStruct((n, _DIM), jnp.float32),
        scratch_shapes=[pltpu.VMEM((_W, 3 * _DIM), jnp.float32)],
    )(nbr, counts, qx, kvx, qtab, ktab)

    return _linear(x, proj_w.T, proj_b)


# speedup vs baseline: 4.7442x; 1.2553x over previous
import functools

import jax
import jax.numpy as jnp
from jax.experimental import pallas as pl
from jax.experimental.pallas import tpu as pltpu

_DIM = 128
_HEADS = 8
_HD = _DIM // _HEADS
_WINDOW_SIZE = 0.6
_QUANT_SIZE = 0.075
_GRID_LEN = int((2 * _WINDOW_SIZE + 0.0001) // _QUANT_SIZE)
_SCALE = _HD ** (-0.5)
_W = 128


def _smem():
    for name in ("SMEM",):
        v = getattr(pltpu, name, None)
        if v is not None:
            return v
    return pltpu.TPUMemorySpace.SMEM


def _pick_block(n, candidates):
    for c in candidates:
        if n % c == 0:
            return c
    return 1


def _linear_kern(x_ref, w_ref, b_ref, o_ref):
    o_ref[...] = (
        jnp.dot(x_ref[...], w_ref[...], preferred_element_type=jnp.float32)
        + b_ref[...]
    )


def _linear(x, w, b):
    n, k = x.shape
    m = w.shape[1]
    r = _pick_block(n, (2000, 1000, 500, 250, 100, 50, 10, 8, 2))
    return pl.pallas_call(
        _linear_kern,
        grid=(n // r,),
        in_specs=[
            pl.BlockSpec((r, k), lambda i: (i, 0)),
            pl.BlockSpec((k, m), lambda i: (0, 0)),
            pl.BlockSpec((1, m), lambda i: (0, 0)),
        ],
        out_specs=pl.BlockSpec((r, m), lambda i: (i, 0)),
        out_shape=jax.ShapeDtypeStruct((n, m), jnp.float32),
    )(x, w, b.reshape(1, m))


def _attn_kern(qb, nbr_ref, cnt_ref, qx_ref, kvx_ref, qtab_ref, ktab_ref,
               o_ref, kvbuf):
    f32 = jnp.float32
    g = (jax.lax.broadcasted_iota(jnp.int32, (_DIM, _HEADS), 0) // _HD
         == jax.lax.broadcasted_iota(jnp.int32, (_DIM, _HEADS), 1)).astype(f32)
    gt = (jax.lax.broadcasted_iota(jnp.int32, (_HEADS, _DIM), 1) // _HD
          == jax.lax.broadcasted_iota(jnp.int32, (_HEADS, _DIM), 0)).astype(f32)
    qtab = qtab_ref[...]
    ktab = ktab_ref[...]
    nbins = 2 * _GRID_LEN
    col = jax.lax.broadcasted_iota(jnp.int32, (_W, 3 * nbins), 1)
    col_c = col // nbins
    col_r = col % nbins
    tvec8 = jax.lax.broadcasted_iota(jnp.int32, (_W, _HEADS), 0)
    tvec128 = jax.lax.broadcasted_iota(jnp.int32, (_W, _DIM), 0)

    for qq in range(qb):
        cnt = jnp.minimum(cnt_ref[qq, 0], _W)

        for t in range(_W):
            j = nbr_ref[qq, t]
            kvbuf[t:t + 1, :] = kvx_ref[pl.ds(j, 1), :]

        qrow = qx_ref[qq:qq + 1, 0:_DIM]
        xq = qx_ref[qq:qq + 1, _DIM:2 * _DIM]
        kb = kvbuf[:, 0:_DIM]
        vb = kvbuf[:, _DIM:2 * _DIM]
        xk = kvbuf[:, 2 * _DIM:3 * _DIM]

        rel = xq - xk
        rel = jnp.round(rel * 100000.0) / 100000.0
        ridx = jnp.floor(
            (rel + (2 * _WINDOW_SIZE - 0.0001)) / _QUANT_SIZE
        ).astype(jnp.int32)
        r0 = jax.lax.slice(ridx, (0, 0), (_W, 1))
        r1 = jax.lax.slice(ridx, (0, 1), (_W, 2))
        r2 = jax.lax.slice(ridx, (0, 2), (_W, 3))
        rsel = jnp.where(col_c == 0, r0, jnp.where(col_c == 1, r1, r2))
        onehot = (rsel == col_r).astype(f32)
        tqs = jnp.dot(onehot, qtab, preferred_element_type=f32)
        tks = jnp.dot(onehot, ktab, preferred_element_type=f32)

        combo = kb * qrow + tqs * qrow + tks * kb
        logits = jnp.dot(combo, g, preferred_element_type=f32)
        valid = tvec8 < cnt
        logits = jnp.where(valid, logits, jnp.float32(-1e30))
        mx = jnp.max(logits, axis=0, keepdims=True)
        e = jnp.where(valid, jnp.exp(logits - mx), 0.0)
        s = jnp.sum(e, axis=0, keepdims=True)
        s = jnp.where(s > 0.0, s, 1.0)
        soft = e / s
        soft128 = jnp.dot(soft, gt, preferred_element_type=f32)
        contrib = jnp.where(tvec128 < cnt, soft128 * vb, 0.0)
        o_ref[qq:qq + 1, :] = jnp.sum(contrib, axis=0, keepdims=True)


def kernel(feats, xyz, index_0, index_1, index_0_offsets, n_max,
           qkv_w, qkv_b, proj_w, proj_b, rel_q_table, rel_k_table):
    n, c = feats.shape
    m = index_1.shape[0]
    nbins = 2 * _GRID_LEN

    qkv = _linear(feats, qkv_w.T, qkv_b)
    qs = qkv[:, 0:_DIM] * _SCALE
    k = qkv[:, _DIM:2 * _DIM]
    v = qkv[:, 2 * _DIM:3 * _DIM]

    xyzp = jnp.concatenate(
        [xyz.astype(jnp.float32), jnp.zeros((n, _DIM - 3), jnp.float32)], axis=1)
    qx = jnp.concatenate([qs, xyzp], axis=1)
    kvx = jnp.concatenate([k, v, xyzp], axis=1)

    offs = index_0_offsets.astype(jnp.int32)
    counts = (offs[1:] - offs[:-1]).reshape(n, 1)
    ar = jnp.arange(_W, dtype=jnp.int32)
    pos = jnp.clip(offs[:-1, None] + ar[None, :], 0, m - 1)
    nbr = jnp.where(ar[None, :] < counts, index_1[pos], 0).astype(jnp.int32)

    qtab = rel_q_table.transpose(3, 0, 1, 2).reshape(3 * nbins, _DIM)
    ktab = rel_k_table.transpose(3, 0, 1, 2).reshape(3 * nbins, _DIM)
    qtab = qtab.astype(jnp.float32)
    ktab = ktab.astype(jnp.float32)

    qb = _pick_block(n, (8, 4, 2))
    smem = _smem()
    x = pl.pallas_call(
        functools.partial(_attn_kern, qb),
        grid=(n // qb,),
        in_specs=[
            pl.BlockSpec((qb, _W), lambda i: (i, 0), memory_space=smem),
            pl.BlockSpec((qb, 1), lambda i: (i, 0), memory_space=smem),
            pl.BlockSpec((qb, 2 * _DIM), lambda i: (i, 0)),
            pl.BlockSpec((n, 3 * _DIM), lambda i: (0, 0)),
            pl.BlockSpec((3 * nbins, _DIM), lambda i: (0, 0)),
            pl.BlockSpec((3 * nbins, _DIM), lambda i: (0, 0)),
        ],
        out_specs=pl.BlockSpec((qb, _DIM), lambda i: (i, 0)),
        out_shape=jax.ShapeDtypeStruct((n, _DIM), jnp.float32),
        scratch_shapes=[pltpu.VMEM((_W, 3 * _DIM), jnp.float32)],
    )(nbr, counts, qx, kvx, qtab, ktab)

    return _linear(x, proj_w.T, proj_b)
